# Initial kernel scaffold; baseline (speedup 1.0000x reference)
import jax
import jax.numpy as jnp
from jax.experimental import pallas as pl


def kernel(x, edge_index_0, edge_index_1, edge_index_2, Ws_0_0, Wd_0_0, atts_0_0, attd_0_0, b_0_0, Ws_0_1, Wd_0_1, atts_0_1, attd_0_1, b_0_1, Ws_0_2, Wd_0_2, atts_0_2, attd_0_2, b_0_2, Ws_1_0, Wd_1_0, atts_1_0, attd_1_0, b_1_0, Ws_1_1, Wd_1_1, atts_1_1, attd_1_1, b_1_1, Ws_1_2, Wd_1_2, atts_1_2, attd_1_2, b_1_2, Ws_2_0, Wd_2_0, atts_2_0, attd_2_0, b_2_0, Ws_2_1, Wd_2_1, atts_2_1, attd_2_1, b_2_1, Ws_2_2, Wd_2_2, atts_2_2, attd_2_2, b_2_2):
    raise NotImplementedError("write your pallas kernel here")


# speedup vs baseline: 8.6814x; 8.6814x over previous
import functools

import jax
import jax.numpy as jnp
from jax import lax
from jax.experimental import pallas as pl
from jax.experimental.pallas import tpu as pltpu
from jax.experimental.pallas import tpu_sc as plsc

N = 10000
NP = 10240
E0 = 160000
E = E0 + N
EPAD = 172032
NSUB = 16
NCH = 96
CK = 112
ROWS_PT = NP // NSUB
RB = 2560
EPS = 1e-16


def _make_sc_edge_pass(hw):
    mesh = plsc.VectorSubcoreMesh(
        core_axis_name="c", subcore_axis_name="s",
        num_cores=2, num_subcores=NSUB)
    grp = CK // 16
    nj = hw // 16

    @functools.partial(
        pl.kernel,
        mesh=mesh,
        compiler_params=pltpu.CompilerParams(
            needs_layout_passes=False, use_tc_tiling_on_sc=False),
        out_type=jax.ShapeDtypeStruct((2, NP, hw), jnp.float32),
        scratch_types=[
            pltpu.VMEM((NCH, CK), jnp.int32),
            pltpu.VMEM((NCH, CK), jnp.int32),
            pltpu.VMEM((NP,), jnp.float32),
            pltpu.VMEM((NP,), jnp.float32),
            pltpu.VMEM((CK,), jnp.float32),
            pltpu.VMEM((CK, hw), jnp.float32),
            pltpu.VMEM((CK, hw), jnp.float32),
            pltpu.VMEM((CK, 16), jnp.float32),
            pltpu.VMEM((64, hw), jnp.float32),
            pltpu.VMEM((64, 16), jnp.float32),
            pltpu.VMEM_SHARED((NP, hw), jnp.float32),
            pltpu.VMEM_SHARED((NP, 16), jnp.float32),
        ],
    )
    def kern(src_hbm, dst_hbm, as_hbm, ad_hbm, hs_hbm, out_hbm,
             src_v, dst_v, as_v, ad_v, ex_v, rows_v, srows_v, sden_v,
             nbuf_v, dbuf_v, acc_sh, den_sh):
        c = lax.axis_index("c")
        s = lax.axis_index("s")
        row0 = s * ROWS_PT

        zv = jnp.zeros((16,), jnp.float32)
        for r in range(64):
            for j in range(nj):
                nbuf_v[r, pl.ds(16 * j, 16)] = zv
            dbuf_v[r, pl.ds(0, 16)] = zv
        for b in range(ROWS_PT // 64):
            pltpu.sync_copy(nbuf_v, acc_sh.at[pl.ds(row0 + 64 * b, 64)])
            pltpu.sync_copy(dbuf_v, den_sh.at[pl.ds(row0 + 64 * b, 64)])

        pltpu.sync_copy(src_hbm.at[c, s], src_v)
        pltpu.sync_copy(dst_hbm.at[s], dst_v)
        pltpu.sync_copy(as_hbm, as_v)
        pltpu.sync_copy(ad_hbm, ad_v)
        plsc.subcore_barrier()

        coff = jnp.full((16,), c * NP, jnp.int32)

        def chunk_body(ci, carry):
            for g in range(grp):
                si = src_v[ci, pl.ds(16 * g, 16)] - coff
                di = dst_v[ci, pl.ds(16 * g, 16)]
                av = (plsc.load_gather(as_v, [si])
                      + plsc.load_gather(ad_v, [di]))
                av = jnp.where(av >= 0.0, av, 0.2 * av)
                av = jnp.minimum(av, 80.0)
                ex_v[pl.ds(16 * g, 16)] = jnp.exp(av)
            pltpu.sync_copy(hs_hbm.at[src_v.at[ci]], rows_v)

            def edge_body(e, carry2):
                ei = jnp.full((16,), e, jnp.int32)
                exv = plsc.load_gather(ex_v, [ei])
                for j in range(nj):
                    srows_v[e, pl.ds(16 * j, 16)] = (
                        rows_v[e, pl.ds(16 * j, 16)] * exv)
                sden_v[e, pl.ds(0, 16)] = exv
                return carry2

            lax.fori_loop(0, CK, edge_body, 0)
            pltpu.sync_copy(srows_v, acc_sh.at[dst_v.at[ci]], add=True)
            pltpu.sync_copy(sden_v, den_sh.at[dst_v.at[ci]], add=True)
            return carry

        lax.fori_loop(0, NCH, chunk_body, 0)
        plsc.subcore_barrier()

        for b in range(ROWS_PT // 64):
            pltpu.sync_copy(acc_sh.at[pl.ds(row0 + 64 * b, 64)], nbuf_v)
            pltpu.sync_copy(den_sh.at[pl.ds(row0 + 64 * b, 64)], dbuf_v)

            def row_body(r, carry2):
                ri = jnp.full((16,), r, jnp.int32)
                dv = plsc.load_gather(
                    dbuf_v, [ri, jnp.zeros((16,), jnp.int32)]) + EPS
                for j in range(nj):
                    nbuf_v[r, pl.ds(16 * j, 16)] = (
                        nbuf_v[r, pl.ds(16 * j, 16)] / dv)
                return carry2

            lax.fori_loop(0, 64, row_body, 0)
            pltpu.sync_copy(nbuf_v, out_hbm.at[c, pl.ds(row0 + 64 * b, 64)])

    return kern


_make_sc_edge_pass = functools.lru_cache(maxsize=None)(_make_sc_edge_pass)


def _prep_body(dout, fused, refs):
    if fused:
        (o0, o1, o2, bs3, wcat, wd, atts, attd, hs_ref, avec_ref) = refs
        i = pl.program_id(0)
        h = (o0[...] + o1[...] + o2[...]) * jnp.float32(1.0 / 3.0) + bs3[...]
        rows = lax.broadcasted_iota(jnp.int32, (RB, 1), 0) + i * RB
        h = jnp.where(rows < N, h, 0.0)
    else:
        (x_ref, wcat, wd, atts, attd, hs_ref, avec_ref) = refs
        h = x_ref[...]
    hs = jnp.dot(h, wcat[...], preferred_element_type=jnp.float32)
    hs_ref[...] = hs
    avs = []
    for t in range(3):
        hst = hs[:, t * dout:(t + 1) * dout]
        avs.append(jnp.sum(hst * atts[t][None, :], axis=-1))
    for t in range(3):
        wdv = jnp.dot(attd[t][None, :], wd[t],
                      preferred_element_type=jnp.float32)
        avs.append(jnp.sum(h * wdv, axis=-1))
    z = jnp.zeros_like(avs[0])
    avec_ref[...] = jnp.stack(avs + [z, z], axis=0)


def _make_tc_prep(dout, fused, win=128):
    grid = (NP // RB,)
    if fused:
        in_specs = [
            pl.BlockSpec((RB, win), lambda i: (i, 0)),
            pl.BlockSpec((RB, win), lambda i: (i, 0)),
            pl.BlockSpec((RB, win), lambda i: (i, 0)),
            pl.BlockSpec((1, win), lambda i: (0, 0)),
            pl.BlockSpec((128, 3 * dout), lambda i: (0, 0)),
            pl.BlockSpec((3, dout, 128), lambda i: (0, 0, 0)),
            pl.BlockSpec((3, dout), lambda i: (0, 0)),
            pl.BlockSpec((3, dout), lambda i: (0, 0)),
        ]
    else:
        in_specs = [
            pl.BlockSpec((RB, 128), lambda i: (i, 0)),
            pl.BlockSpec((128, 3 * dout), lambda i: (0, 0)),
            pl.BlockSpec((3, dout, 128), lambda i: (0, 0, 0)),
            pl.BlockSpec((3, dout), lambda i: (0, 0)),
            pl.BlockSpec((3, dout), lambda i: (0, 0)),
        ]
    out_specs = [
        pl.BlockSpec((RB, 3 * dout), lambda i: (i, 0)),
        pl.BlockSpec((8, RB), lambda i: (0, i)),
    ]
    out_shape = [
        jax.ShapeDtypeStruct((NP, 3 * dout), jnp.float32),
        jax.ShapeDtypeStruct((8, NP), jnp.float32),
    ]
    return pl.pallas_call(
        lambda *refs: _prep_body(dout, fused, refs),
        grid=grid, in_specs=in_specs, out_specs=out_specs,
        out_shape=out_shape)


def _final_body(o0, o1, o2, bs3, out_ref):
    out_ref[...] = (o0[...] + o1[...] + o2[...]) * jnp.float32(1.0 / 3.0) \
        + bs3[...]


_tc_final = pl.pallas_call(
    _final_body,
    grid=(NP // RB,),
    in_specs=[
        pl.BlockSpec((RB, 256), lambda i: (i, 0)),
        pl.BlockSpec((RB, 256), lambda i: (i, 0)),
        pl.BlockSpec((RB, 256), lambda i: (i, 0)),
        pl.BlockSpec((1, 256), lambda i: (0, 0)),
    ],
    out_specs=pl.BlockSpec((RB, 256), lambda i: (i, 0)),
    out_shape=jax.ShapeDtypeStruct((NP, 256), jnp.float32),
)


def _prep_edges(ei):
    s0, d0 = ei[0], ei[1]
    d0 = jnp.where(s0 != d0, d0, N)
    loop = jnp.arange(N, dtype=jnp.int32)
    src = jnp.concatenate([s0, loop])
    dst = jnp.concatenate([d0, loop])
    src = jnp.pad(src, (0, EPAD - E))
    dst = jnp.pad(dst, (0, EPAD - E), constant_values=N)
    src2 = jnp.stack([src, src + NP]).reshape(2, NSUB, NCH, CK)
    dst2 = dst.reshape(NSUB, NCH, CK)
    return src2, dst2


def _halves(hs_t, hw):
    return jnp.stack([hs_t[:, :hw], hs_t[:, hw:]]).reshape(2 * NP, hw)


def kernel(x, edge_index_0, edge_index_1, edge_index_2,
           Ws_0_0, Wd_0_0, atts_0_0, attd_0_0, b_0_0,
           Ws_0_1, Wd_0_1, atts_0_1, attd_0_1, b_0_1,
           Ws_0_2, Wd_0_2, atts_0_2, attd_0_2, b_0_2,
           Ws_1_0, Wd_1_0, atts_1_0, attd_1_0, b_1_0,
           Ws_1_1, Wd_1_1, atts_1_1, attd_1_1, b_1_1,
           Ws_1_2, Wd_1_2, atts_1_2, attd_1_2, b_1_2,
           Ws_2_0, Wd_2_0, atts_2_0, attd_2_0, b_2_0,
           Ws_2_1, Wd_2_1, atts_2_1, attd_2_1, b_2_1,
           Ws_2_2, Wd_2_2, atts_2_2, attd_2_2, b_2_2):
    p = dict(locals())
    edges = [_prep_edges(e) for e in (edge_index_0, edge_index_1, edge_index_2)]

    xp = jnp.pad(x, ((0, NP - N), (0, 0)))

    def layer_weights(l):
        dout = 256 if l == 2 else 128
        wcat = jnp.concatenate(
            [p[f"Ws_{l}_{t}"].T for t in range(3)], axis=1)
        wd = jnp.stack([p[f"Wd_{l}_{t}"] for t in range(3)])
        atts = jnp.stack([p[f"atts_{l}_{t}"] for t in range(3)])
        attd = jnp.stack([p[f"attd_{l}_{t}"] for t in range(3)])
        bs3 = ((p[f"b_{l}_0"] + p[f"b_{l}_1"] + p[f"b_{l}_2"])
               * jnp.float32(1.0 / 3.0))[None, :]
        return dout, wcat, wd, atts, attd, bs3

    outs = None
    prev_bs3 = None
    for l in range(3):
        dout, wcat, wd, atts, attd, bs3 = layer_weights(l)
        sc_pass = _make_sc_edge_pass(64)
        if l == 0:
            hs_cat, avec = _make_tc_prep(dout, False)(xp, wcat, wd, atts, attd)
        else:
            hs_cat, avec = _make_tc_prep(dout, True)(
                outs[0], outs[1], outs[2], prev_bs3, wcat, wd, atts, attd)
        outs = []
        for t in range(3):
            pieces = []
            for h in range(dout // 128):
                hs_t = hs_cat[:, t * dout + 128 * h:t * dout + 128 * (h + 1)]
                hsf = _halves(hs_t, 64)
                norm = sc_pass(edges[t][0], edges[t][1],
                               avec[t], avec[3 + t], hsf)
                pieces += [norm[0], norm[1]]
            outs.append(jnp.concatenate(pieces, axis=1))
        prev_bs3 = bs3
    out = _tc_final(outs[0], outs[1], outs[2], prev_bs3)
    return out[:N]


# speedup vs baseline: 13.0443x; 1.5026x over previous
import functools

import jax
import jax.numpy as jnp
from jax import lax
from jax.experimental import pallas as pl
from jax.experimental.pallas import tpu as pltpu
from jax.experimental.pallas import tpu_sc as plsc

N = 10000
NP = 10240
E0 = 160000
E = E0 + N
EPAD = 172032
NSUB = 16
NCH = 96
CK = 112
ROWS_PT = NP // NSUB
RB = 2560
EPS = 1e-16


def _make_sc_edge_pass(hw):
    mesh = plsc.VectorSubcoreMesh(
        core_axis_name="c", subcore_axis_name="s",
        num_cores=2, num_subcores=NSUB)
    grp = CK // 16
    nj = hw // 16

    @functools.partial(
        pl.kernel,
        mesh=mesh,
        compiler_params=pltpu.CompilerParams(
            needs_layout_passes=False, use_tc_tiling_on_sc=False),
        out_type=jax.ShapeDtypeStruct((2, NP, hw), jnp.float32),
        scratch_types=[
            pltpu.VMEM((NCH, CK), jnp.int32),
            pltpu.VMEM((NCH, CK), jnp.int32),
            pltpu.VMEM((NP,), jnp.float32),
            pltpu.VMEM((NP,), jnp.float32),
            pltpu.VMEM((CK,), jnp.float32),
            pltpu.VMEM((2, CK, hw), jnp.float32),
            pltpu.VMEM((2, CK, hw), jnp.float32),
            pltpu.VMEM((2, CK, 16), jnp.float32),
            pltpu.VMEM((64, hw), jnp.float32),
            pltpu.VMEM((64, 16), jnp.float32),
            pltpu.VMEM_SHARED((NP, hw), jnp.float32),
            pltpu.VMEM_SHARED((NP, 16), jnp.float32),
            pltpu.SemaphoreType.DMA,
            pltpu.SemaphoreType.DMA,
        ],
    )
    def kern(src_hbm, dst_hbm, as_hbm, ad_hbm, hs_hbm, out_hbm,
             src_v, dst_v, as_v, ad_v, ex_v, rows_v, srows_v, sden_v,
             nbuf_v, dbuf_v, acc_sh, den_sh, gsem, ssem):
        c = lax.axis_index("c")
        s = lax.axis_index("s")
        row0 = s * ROWS_PT

        zv = jnp.zeros((16,), jnp.float32)
        for r in range(64):
            for j in range(nj):
                nbuf_v[r, pl.ds(16 * j, 16)] = zv
            dbuf_v[r, pl.ds(0, 16)] = zv
        for b in range(ROWS_PT // 64):
            pltpu.sync_copy(nbuf_v, acc_sh.at[pl.ds(row0 + 64 * b, 64)])
            pltpu.sync_copy(dbuf_v, den_sh.at[pl.ds(row0 + 64 * b, 64)])

        pltpu.sync_copy(src_hbm.at[c, s], src_v)
        pltpu.sync_copy(dst_hbm.at[s], dst_v)
        pltpu.sync_copy(as_hbm, as_v)
        pltpu.sync_copy(ad_hbm, ad_v)
        plsc.subcore_barrier()

        coff = jnp.full((16,), c * NP, jnp.int32)

        pltpu.async_copy(hs_hbm.at[src_v.at[0]], rows_v.at[0], gsem)

        def pair_body(i, carry):
            for k in range(2):
                ci = 2 * i + k

                @pl.when(ci + 1 < NCH)
                def _():
                    pltpu.async_copy(hs_hbm.at[src_v.at[ci + 1]],
                                     rows_v.at[1 - k], gsem)

                for g in range(grp):
                    si = src_v[ci, pl.ds(16 * g, 16)] - coff
                    di = dst_v[ci, pl.ds(16 * g, 16)]
                    av = (plsc.load_gather(as_v, [si])
                          + plsc.load_gather(ad_v, [di]))
                    av = jnp.where(av >= 0.0, av, 0.2 * av)
                    av = jnp.minimum(av, 80.0)
                    ex_v[pl.ds(16 * g, 16)] = jnp.exp(av)

                pltpu.make_async_copy(
                    hs_hbm.at[pl.ds(0, CK)], rows_v.at[k], gsem).wait()

                @pl.when(ci >= 2)
                def _():
                    pltpu.make_async_copy(
                        srows_v.at[k], acc_sh.at[pl.ds(0, CK)], ssem).wait()
                    pltpu.make_async_copy(
                        sden_v.at[k], den_sh.at[pl.ds(0, CK)], ssem).wait()

                def edge_body(e, carry2):
                    ei = jnp.full((16,), e, jnp.int32)
                    exv = plsc.load_gather(ex_v, [ei])
                    for j in range(nj):
                        srows_v[k, e, pl.ds(16 * j, 16)] = (
                            rows_v[k, e, pl.ds(16 * j, 16)] * exv)
                    sden_v[k, e, pl.ds(0, 16)] = exv
                    return carry2

                lax.fori_loop(0, CK, edge_body, 0)
                pltpu.async_copy(srows_v.at[k], acc_sh.at[dst_v.at[ci]],
                                 ssem, add=True)
                pltpu.async_copy(sden_v.at[k], den_sh.at[dst_v.at[ci]],
                                 ssem, add=True)
            return carry

        lax.fori_loop(0, NCH // 2, pair_body, 0)
        for k in range(2):
            pltpu.make_async_copy(
                srows_v.at[k], acc_sh.at[pl.ds(0, CK)], ssem).wait()
            pltpu.make_async_copy(
                sden_v.at[k], den_sh.at[pl.ds(0, CK)], ssem).wait()
        plsc.subcore_barrier()

        for b in range(ROWS_PT // 64):
            pltpu.sync_copy(acc_sh.at[pl.ds(row0 + 64 * b, 64)], nbuf_v)
            pltpu.sync_copy(den_sh.at[pl.ds(row0 + 64 * b, 64)], dbuf_v)

            def row_body(r, carry2):
                ri = jnp.full((16,), r, jnp.int32)
                dv = plsc.load_gather(
                    dbuf_v, [ri, jnp.zeros((16,), jnp.int32)]) + EPS
                for j in range(nj):
                    nbuf_v[r, pl.ds(16 * j, 16)] = (
                        nbuf_v[r, pl.ds(16 * j, 16)] / dv)
                return carry2

            lax.fori_loop(0, 64, row_body, 0)
            pltpu.sync_copy(nbuf_v, out_hbm.at[c, pl.ds(row0 + 64 * b, 64)])

    return kern


_make_sc_edge_pass = functools.lru_cache(maxsize=None)(_make_sc_edge_pass)


def _prep_body(dout, fused, refs):
    if fused:
        (o0, o1, o2, bs3, wcat, wd, atts, attd, hs_ref, avec_ref) = refs
        i = pl.program_id(0)
        h = (o0[...] + o1[...] + o2[...]) * jnp.float32(1.0 / 3.0) + bs3[...]
        rows = lax.broadcasted_iota(jnp.int32, (RB, 1), 0) + i * RB
        h = jnp.where(rows < N, h, 0.0)
    else:
        (x_ref, wcat, wd, atts, attd, hs_ref, avec_ref) = refs
        h = x_ref[...]
    hs = jnp.dot(h, wcat[...], preferred_element_type=jnp.float32)
    hs_ref[...] = hs
    avs = []
    for t in range(3):
        hst = hs[:, t * dout:(t + 1) * dout]
        avs.append(jnp.sum(hst * atts[t][None, :], axis=-1))
    for t in range(3):
        wdv = jnp.dot(attd[t][None, :], wd[t],
                      preferred_element_type=jnp.float32)
        avs.append(jnp.sum(h * wdv, axis=-1))
    z = jnp.zeros_like(avs[0])
    avec_ref[...] = jnp.stack(avs + [z, z], axis=0)


def _make_tc_prep(dout, fused, win=128):
    grid = (NP // RB,)
    if fused:
        in_specs = [
            pl.BlockSpec((RB, win), lambda i: (i, 0)),
            pl.BlockSpec((RB, win), lambda i: (i, 0)),
            pl.BlockSpec((RB, win), lambda i: (i, 0)),
            pl.BlockSpec((1, win), lambda i: (0, 0)),
            pl.BlockSpec((128, 3 * dout), lambda i: (0, 0)),
            pl.BlockSpec((3, dout, 128), lambda i: (0, 0, 0)),
            pl.BlockSpec((3, dout), lambda i: (0, 0)),
            pl.BlockSpec((3, dout), lambda i: (0, 0)),
        ]
    else:
        in_specs = [
            pl.BlockSpec((RB, 128), lambda i: (i, 0)),
            pl.BlockSpec((128, 3 * dout), lambda i: (0, 0)),
            pl.BlockSpec((3, dout, 128), lambda i: (0, 0, 0)),
            pl.BlockSpec((3, dout), lambda i: (0, 0)),
            pl.BlockSpec((3, dout), lambda i: (0, 0)),
        ]
    out_specs = [
        pl.BlockSpec((RB, 3 * dout), lambda i: (i, 0)),
        pl.BlockSpec((8, RB), lambda i: (0, i)),
    ]
    out_shape = [
        jax.ShapeDtypeStruct((NP, 3 * dout), jnp.float32),
        jax.ShapeDtypeStruct((8, NP), jnp.float32),
    ]
    return pl.pallas_call(
        lambda *refs: _prep_body(dout, fused, refs),
        grid=grid, in_specs=in_specs, out_specs=out_specs,
        out_shape=out_shape)


def _final_body(o0, o1, o2, bs3, out_ref):
    out_ref[...] = (o0[...] + o1[...] + o2[...]) * jnp.float32(1.0 / 3.0) \
        + bs3[...]


_tc_final = pl.pallas_call(
    _final_body,
    grid=(NP // RB,),
    in_specs=[
        pl.BlockSpec((RB, 256), lambda i: (i, 0)),
        pl.BlockSpec((RB, 256), lambda i: (i, 0)),
        pl.BlockSpec((RB, 256), lambda i: (i, 0)),
        pl.BlockSpec((1, 256), lambda i: (0, 0)),
    ],
    out_specs=pl.BlockSpec((RB, 256), lambda i: (i, 0)),
    out_shape=jax.ShapeDtypeStruct((NP, 256), jnp.float32),
)


def _prep_edges(ei):
    s0, d0 = ei[0], ei[1]
    d0 = jnp.where(s0 != d0, d0, N)
    loop = jnp.arange(N, dtype=jnp.int32)
    src = jnp.concatenate([s0, loop])
    dst = jnp.concatenate([d0, loop])
    src = jnp.pad(src, (0, EPAD - E))
    dst = jnp.pad(dst, (0, EPAD - E), constant_values=N)
    src2 = jnp.stack([src, src + NP]).reshape(2, NSUB, NCH, CK)
    dst2 = dst.reshape(NSUB, NCH, CK)
    return src2, dst2


def _halves(hs_t, hw):
    return jnp.stack([hs_t[:, :hw], hs_t[:, hw:]]).reshape(2 * NP, hw)


def kernel(x, edge_index_0, edge_index_1, edge_index_2,
           Ws_0_0, Wd_0_0, atts_0_0, attd_0_0, b_0_0,
           Ws_0_1, Wd_0_1, atts_0_1, attd_0_1, b_0_1,
           Ws_0_2, Wd_0_2, atts_0_2, attd_0_2, b_0_2,
           Ws_1_0, Wd_1_0, atts_1_0, attd_1_0, b_1_0,
           Ws_1_1, Wd_1_1, atts_1_1, attd_1_1, b_1_1,
           Ws_1_2, Wd_1_2, atts_1_2, attd_1_2, b_1_2,
           Ws_2_0, Wd_2_0, atts_2_0, attd_2_0, b_2_0,
           Ws_2_1, Wd_2_1, atts_2_1, attd_2_1, b_2_1,
           Ws_2_2, Wd_2_2, atts_2_2, attd_2_2, b_2_2):
    p = dict(locals())
    edges = [_prep_edges(e) for e in (edge_index_0, edge_index_1, edge_index_2)]

    xp = jnp.pad(x, ((0, NP - N), (0, 0)))

    def layer_weights(l):
        dout = 256 if l == 2 else 128
        wcat = jnp.concatenate(
            [p[f"Ws_{l}_{t}"].T for t in range(3)], axis=1)
        wd = jnp.stack([p[f"Wd_{l}_{t}"] for t in range(3)])
        atts = jnp.stack([p[f"atts_{l}_{t}"] for t in range(3)])
        attd = jnp.stack([p[f"attd_{l}_{t}"] for t in range(3)])
        bs3 = ((p[f"b_{l}_0"] + p[f"b_{l}_1"] + p[f"b_{l}_2"])
               * jnp.float32(1.0 / 3.0))[None, :]
        return dout, wcat, wd, atts, attd, bs3

    outs = None
    prev_bs3 = None
    for l in range(3):
        dout, wcat, wd, atts, attd, bs3 = layer_weights(l)
        sc_pass = _make_sc_edge_pass(64)
        if l == 0:
            hs_cat, avec = _make_tc_prep(dout, False)(xp, wcat, wd, atts, attd)
        else:
            hs_cat, avec = _make_tc_prep(dout, True)(
                outs[0], outs[1], outs[2], prev_bs3, wcat, wd, atts, attd)
        outs = []
        for t in range(3):
            pieces = []
            for h in range(dout // 128):
                hs_t = hs_cat[:, t * dout + 128 * h:t * dout + 128 * (h + 1)]
                hsf = _halves(hs_t, 64)
                norm = sc_pass(edges[t][0], edges[t][1],
                               avec[t], avec[3 + t], hsf)
                pieces += [norm[0], norm[1]]
            outs.append(jnp.concatenate(pieces, axis=1))
        prev_bs3 = bs3
    out = _tc_final(outs[0], outs[1], outs[2], prev_bs3)
    return out[:N]


# speedup vs baseline: 22.8530x; 1.7520x over previous
import functools

import jax
import jax.numpy as jnp
from jax import lax
from jax.experimental import pallas as pl
from jax.experimental.pallas import tpu as pltpu
from jax.experimental.pallas import tpu_sc as plsc

N = 10000
NP = 10240
E0 = 160000
E = E0 + N
EPAD = 172032
NSUB = 16
NCH = 96
CK = 112
ROWS_PT = NP // NSUB
RB = 2560
EPS = 1e-16


def _make_sc_edge_pass(hw):
    mesh = plsc.VectorSubcoreMesh(
        core_axis_name="c", subcore_axis_name="s",
        num_cores=2, num_subcores=NSUB)
    grp = CK // 16
    nj = hw // 16

    @functools.partial(
        pl.kernel,
        mesh=mesh,
        compiler_params=pltpu.CompilerParams(
            needs_layout_passes=False, use_tc_tiling_on_sc=False),
        out_type=jax.ShapeDtypeStruct((2, NP, hw), jnp.float32),
        scratch_types=[
            pltpu.VMEM((NCH, CK), jnp.int32),
            pltpu.VMEM((NCH, CK), jnp.int32),
            pltpu.VMEM((NP,), jnp.float32),
            pltpu.VMEM((NP,), jnp.float32),
            pltpu.VMEM((CK,), jnp.float32),
            pltpu.VMEM((2, CK, hw), jnp.float32),
            pltpu.VMEM((2, CK, hw), jnp.float32),
            pltpu.VMEM((2, CK, 16), jnp.float32),
            pltpu.VMEM((64, hw), jnp.float32),
            pltpu.VMEM((64, 16), jnp.float32),
            pltpu.VMEM_SHARED((NP, hw), jnp.float32),
            pltpu.VMEM_SHARED((NP, 16), jnp.float32),
            pltpu.SemaphoreType.DMA,
            pltpu.SemaphoreType.DMA,
        ],
    )
    def kern(src_hbm, dst_hbm, as_hbm, ad_hbm, hs_hbm, out_hbm,
             src_v, dst_v, as_v, ad_v, ex_v, rows_v, srows_v, sden_v,
             nbuf_v, dbuf_v, acc_sh, den_sh, gsem, ssem):
        c = lax.axis_index("c")
        s = lax.axis_index("s")
        row0 = s * ROWS_PT

        zv = jnp.zeros((16,), jnp.float32)
        for r in range(64):
            for j in range(nj):
                nbuf_v[r, pl.ds(16 * j, 16)] = zv
            dbuf_v[r, pl.ds(0, 16)] = zv
        for b in range(ROWS_PT // 64):
            pltpu.sync_copy(nbuf_v, acc_sh.at[pl.ds(row0 + 64 * b, 64)])
            pltpu.sync_copy(dbuf_v, den_sh.at[pl.ds(row0 + 64 * b, 64)])

        pltpu.sync_copy(src_hbm.at[c, s], src_v)
        pltpu.sync_copy(dst_hbm.at[s], dst_v)
        pltpu.sync_copy(as_hbm, as_v)
        pltpu.sync_copy(ad_hbm, ad_v)
        plsc.subcore_barrier()

        coff = jnp.full((16,), c * NP, jnp.int32)

        pltpu.async_copy(hs_hbm.at[src_v.at[0]], rows_v.at[0], gsem)

        def pair_body(i, carry):
            for k in range(2):
                ci = 2 * i + k

                @pl.when(ci + 1 < NCH)
                def _():
                    pltpu.async_copy(hs_hbm.at[src_v.at[ci + 1]],
                                     rows_v.at[1 - k], gsem)

                for g in range(grp):
                    si = src_v[ci, pl.ds(16 * g, 16)] - coff
                    di = dst_v[ci, pl.ds(16 * g, 16)]
                    av = (plsc.load_gather(as_v, [si])
                          + plsc.load_gather(ad_v, [di]))
                    av = jnp.where(av >= 0.0, av, 0.2 * av)
                    av = jnp.minimum(av, 80.0)
                    ex_v[pl.ds(16 * g, 16)] = jnp.exp(av)

                pltpu.make_async_copy(
                    hs_hbm.at[pl.ds(0, CK)], rows_v.at[k], gsem).wait()

                @pl.when(ci >= 2)
                def _():
                    pltpu.make_async_copy(
                        srows_v.at[k], acc_sh.at[pl.ds(0, CK)], ssem).wait()
                    pltpu.make_async_copy(
                        sden_v.at[k], den_sh.at[pl.ds(0, CK)], ssem).wait()

                @plsc.parallel_loop(0, CK, 1, unroll=8)
                def _(e):
                    ei = jnp.full((16,), e, jnp.int32)
                    exv = plsc.load_gather(ex_v, [ei])
                    for j in range(nj):
                        srows_v[k, e, pl.ds(16 * j, 16)] = (
                            rows_v[k, e, pl.ds(16 * j, 16)] * exv)
                    sden_v[k, e, pl.ds(0, 16)] = exv
                pltpu.async_copy(srows_v.at[k], acc_sh.at[dst_v.at[ci]],
                                 ssem, add=True)
                pltpu.async_copy(sden_v.at[k], den_sh.at[dst_v.at[ci]],
                                 ssem, add=True)
            return carry

        lax.fori_loop(0, NCH // 2, pair_body, 0)
        for k in range(2):
            pltpu.make_async_copy(
                srows_v.at[k], acc_sh.at[pl.ds(0, CK)], ssem).wait()
            pltpu.make_async_copy(
                sden_v.at[k], den_sh.at[pl.ds(0, CK)], ssem).wait()
        plsc.subcore_barrier()

        for b in range(ROWS_PT // 64):
            pltpu.sync_copy(acc_sh.at[pl.ds(row0 + 64 * b, 64)], nbuf_v)
            pltpu.sync_copy(den_sh.at[pl.ds(row0 + 64 * b, 64)], dbuf_v)

            @plsc.parallel_loop(0, 64, 1, unroll=8)
            def _(r):
                ri = jnp.full((16,), r, jnp.int32)
                dv = plsc.load_gather(
                    dbuf_v, [ri, jnp.zeros((16,), jnp.int32)]) + EPS
                for j in range(nj):
                    nbuf_v[r, pl.ds(16 * j, 16)] = (
                        nbuf_v[r, pl.ds(16 * j, 16)] / dv)
            pltpu.sync_copy(nbuf_v, out_hbm.at[c, pl.ds(row0 + 64 * b, 64)])

    return kern


_make_sc_edge_pass = functools.lru_cache(maxsize=None)(_make_sc_edge_pass)


def _prep_body(dout, fused, refs):
    if fused:
        (o0, o1, o2, bs3, wcat, wd, atts, attd, hs_ref, avec_ref) = refs
        i = pl.program_id(0)
        h = (o0[...] + o1[...] + o2[...]) * jnp.float32(1.0 / 3.0) + bs3[...]
        rows = lax.broadcasted_iota(jnp.int32, (RB, 1), 0) + i * RB
        h = jnp.where(rows < N, h, 0.0)
    else:
        (x_ref, wcat, wd, atts, attd, hs_ref, avec_ref) = refs
        h = x_ref[...]
    hs = jnp.dot(h, wcat[...], preferred_element_type=jnp.float32)
    hs_ref[...] = hs
    avs = []
    for t in range(3):
        hst = hs[:, t * dout:(t + 1) * dout]
        avs.append(jnp.sum(hst * atts[t][None, :], axis=-1))
    for t in range(3):
        wdv = jnp.dot(attd[t][None, :], wd[t],
                      preferred_element_type=jnp.float32)
        avs.append(jnp.sum(h * wdv, axis=-1))
    z = jnp.zeros_like(avs[0])
    avec_ref[...] = jnp.stack(avs + [z, z], axis=0)


def _make_tc_prep(dout, fused, win=128):
    grid = (NP // RB,)
    if fused:
        in_specs = [
            pl.BlockSpec((RB, win), lambda i: (i, 0)),
            pl.BlockSpec((RB, win), lambda i: (i, 0)),
            pl.BlockSpec((RB, win), lambda i: (i, 0)),
            pl.BlockSpec((1, win), lambda i: (0, 0)),
            pl.BlockSpec((128, 3 * dout), lambda i: (0, 0)),
            pl.BlockSpec((3, dout, 128), lambda i: (0, 0, 0)),
            pl.BlockSpec((3, dout), lambda i: (0, 0)),
            pl.BlockSpec((3, dout), lambda i: (0, 0)),
        ]
    else:
        in_specs = [
            pl.BlockSpec((RB, 128), lambda i: (i, 0)),
            pl.BlockSpec((128, 3 * dout), lambda i: (0, 0)),
            pl.BlockSpec((3, dout, 128), lambda i: (0, 0, 0)),
            pl.BlockSpec((3, dout), lambda i: (0, 0)),
            pl.BlockSpec((3, dout), lambda i: (0, 0)),
        ]
    out_specs = [
        pl.BlockSpec((RB, 3 * dout), lambda i: (i, 0)),
        pl.BlockSpec((8, RB), lambda i: (0, i)),
    ]
    out_shape = [
        jax.ShapeDtypeStruct((NP, 3 * dout), jnp.float32),
        jax.ShapeDtypeStruct((8, NP), jnp.float32),
    ]
    return pl.pallas_call(
        lambda *refs: _prep_body(dout, fused, refs),
        grid=grid, in_specs=in_specs, out_specs=out_specs,
        out_shape=out_shape)


def _final_body(o0, o1, o2, bs3, out_ref):
    out_ref[...] = (o0[...] + o1[...] + o2[...]) * jnp.float32(1.0 / 3.0) \
        + bs3[...]


_tc_final = pl.pallas_call(
    _final_body,
    grid=(NP // RB,),
    in_specs=[
        pl.BlockSpec((RB, 256), lambda i: (i, 0)),
        pl.BlockSpec((RB, 256), lambda i: (i, 0)),
        pl.BlockSpec((RB, 256), lambda i: (i, 0)),
        pl.BlockSpec((1, 256), lambda i: (0, 0)),
    ],
    out_specs=pl.BlockSpec((RB, 256), lambda i: (i, 0)),
    out_shape=jax.ShapeDtypeStruct((NP, 256), jnp.float32),
)


def _prep_edges(ei):
    s0, d0 = ei[0], ei[1]
    d0 = jnp.where(s0 != d0, d0, N)
    loop = jnp.arange(N, dtype=jnp.int32)
    src = jnp.concatenate([s0, loop])
    dst = jnp.concatenate([d0, loop])
    src = jnp.pad(src, (0, EPAD - E))
    dst = jnp.pad(dst, (0, EPAD - E), constant_values=N)
    src2 = jnp.stack([src, src + NP]).reshape(2, NSUB, NCH, CK)
    dst2 = dst.reshape(NSUB, NCH, CK)
    return src2, dst2


def _halves(hs_t, hw):
    return jnp.stack([hs_t[:, :hw], hs_t[:, hw:]]).reshape(2 * NP, hw)


def kernel(x, edge_index_0, edge_index_1, edge_index_2,
           Ws_0_0, Wd_0_0, atts_0_0, attd_0_0, b_0_0,
           Ws_0_1, Wd_0_1, atts_0_1, attd_0_1, b_0_1,
           Ws_0_2, Wd_0_2, atts_0_2, attd_0_2, b_0_2,
           Ws_1_0, Wd_1_0, atts_1_0, attd_1_0, b_1_0,
           Ws_1_1, Wd_1_1, atts_1_1, attd_1_1, b_1_1,
           Ws_1_2, Wd_1_2, atts_1_2, attd_1_2, b_1_2,
           Ws_2_0, Wd_2_0, atts_2_0, attd_2_0, b_2_0,
           Ws_2_1, Wd_2_1, atts_2_1, attd_2_1, b_2_1,
           Ws_2_2, Wd_2_2, atts_2_2, attd_2_2, b_2_2):
    p = dict(locals())
    edges = [_prep_edges(e) for e in (edge_index_0, edge_index_1, edge_index_2)]

    xp = jnp.pad(x, ((0, NP - N), (0, 0)))

    def layer_weights(l):
        dout = 256 if l == 2 else 128
        wcat = jnp.concatenate(
            [p[f"Ws_{l}_{t}"].T for t in range(3)], axis=1)
        wd = jnp.stack([p[f"Wd_{l}_{t}"] for t in range(3)])
        atts = jnp.stack([p[f"atts_{l}_{t}"] for t in range(3)])
        attd = jnp.stack([p[f"attd_{l}_{t}"] for t in range(3)])
        bs3 = ((p[f"b_{l}_0"] + p[f"b_{l}_1"] + p[f"b_{l}_2"])
               * jnp.float32(1.0 / 3.0))[None, :]
        return dout, wcat, wd, atts, attd, bs3

    outs = None
    prev_bs3 = None
    for l in range(3):
        dout, wcat, wd, atts, attd, bs3 = layer_weights(l)
        sc_pass = _make_sc_edge_pass(64)
        if l == 0:
            hs_cat, avec = _make_tc_prep(dout, False)(xp, wcat, wd, atts, attd)
        else:
            hs_cat, avec = _make_tc_prep(dout, True)(
                outs[0], outs[1], outs[2], prev_bs3, wcat, wd, atts, attd)
        outs = []
        for t in range(3):
            pieces = []
            for h in range(dout // 128):
                hs_t = hs_cat[:, t * dout + 128 * h:t * dout + 128 * (h + 1)]
                hsf = _halves(hs_t, 64)
                norm = sc_pass(edges[t][0], edges[t][1],
                               avec[t], avec[3 + t], hsf)
                pieces += [norm[0], norm[1]]
            outs.append(jnp.concatenate(pieces, axis=1))
        prev_bs3 = bs3
    out = _tc_final(outs[0], outs[1], outs[2], prev_bs3)
    return out[:N]


# speedup vs baseline: 23.6816x; 1.0363x over previous
import functools

import jax
import jax.numpy as jnp
from jax import lax
from jax.experimental import pallas as pl
from jax.experimental.pallas import tpu as pltpu
from jax.experimental.pallas import tpu_sc as plsc

N = 10000
NP = 10240
E0 = 160000
E = E0 + N
EPAD = 172032
NSUB = 16
NCH = 96
CK = 112
ROWS_PT = NP // NSUB
RB = 2560
EPS = 1e-16


def _make_sc_edge_pass(hw):
    mesh = plsc.VectorSubcoreMesh(
        core_axis_name="c", subcore_axis_name="s",
        num_cores=2, num_subcores=NSUB)
    grp = CK // 16
    nj = hw // 16

    @functools.partial(
        pl.kernel,
        mesh=mesh,
        compiler_params=pltpu.CompilerParams(
            needs_layout_passes=False, use_tc_tiling_on_sc=False),
        out_type=jax.ShapeDtypeStruct((2, NP, hw), jnp.float32),
        scratch_types=[
            pltpu.VMEM((NCH, CK), jnp.int32),
            pltpu.VMEM((NCH, CK), jnp.int32),
            pltpu.VMEM((NP,), jnp.float32),
            pltpu.VMEM((NP,), jnp.float32),
            pltpu.VMEM((CK,), jnp.float32),
            pltpu.VMEM((3, CK, hw), jnp.float32),
            pltpu.VMEM((2, CK, 16), jnp.float32),
            pltpu.VMEM((2, 64, hw), jnp.float32),
            pltpu.VMEM((2, 64, 16), jnp.float32),
            pltpu.VMEM_SHARED((NP, hw), jnp.float32),
            pltpu.VMEM_SHARED((NP, 16), jnp.float32),
            pltpu.SemaphoreType.DMA,
            pltpu.SemaphoreType.DMA,
        ],
    )
    def kern(src_hbm, dst_hbm, as_hbm, ad_hbm, hs_hbm, out_hbm,
             src_v, dst_v, as_v, ad_v, ex_v, rows_v, sden_v,
             nbuf_v, dbuf_v, acc_sh, den_sh, gsem, ssem):
        c = lax.axis_index("c")
        s = lax.axis_index("s")
        row0 = s * ROWS_PT

        zv = jnp.zeros((16,), jnp.float32)

        @plsc.parallel_loop(0, 64, 1, unroll=8)
        def _(r):
            for j in range(nj):
                nbuf_v[0, r, pl.ds(16 * j, 16)] = zv
            dbuf_v[0, r, pl.ds(0, 16)] = zv
        for b in range(ROWS_PT // 64):
            pltpu.async_copy(nbuf_v.at[0],
                             acc_sh.at[pl.ds(row0 + 64 * b, 64)], gsem)
            pltpu.async_copy(dbuf_v.at[0],
                             den_sh.at[pl.ds(row0 + 64 * b, 64)], gsem)

        pltpu.sync_copy(src_hbm.at[c, s], src_v)
        pltpu.sync_copy(dst_hbm.at[s], dst_v)
        pltpu.sync_copy(as_hbm, as_v)
        pltpu.sync_copy(ad_hbm, ad_v)
        for b in range(ROWS_PT // 64):
            pltpu.make_async_copy(
                nbuf_v.at[0], acc_sh.at[pl.ds(0, 64)], gsem).wait()
            pltpu.make_async_copy(
                dbuf_v.at[0], den_sh.at[pl.ds(0, 64)], gsem).wait()
        plsc.subcore_barrier()

        coff = jnp.full((16,), c * NP, jnp.int32)

        pltpu.async_copy(hs_hbm.at[src_v.at[0]], rows_v.at[0], gsem)

        def six_body(i, carry):
            for k6 in range(6):
                ci = 6 * i + k6
                rb = k6 % 3
                k2 = k6 % 2

                @pl.when(ci >= 2)
                def _():
                    pltpu.make_async_copy(
                        rows_v.at[(k6 + 1) % 3], acc_sh.at[pl.ds(0, CK)],
                        ssem).wait()
                    pltpu.make_async_copy(
                        sden_v.at[k2], den_sh.at[pl.ds(0, CK)], ssem).wait()

                @pl.when(ci + 1 < NCH)
                def _():
                    pltpu.async_copy(hs_hbm.at[src_v.at[ci + 1]],
                                     rows_v.at[(k6 + 1) % 3], gsem)

                for g in range(grp):
                    si = src_v[ci, pl.ds(16 * g, 16)] - coff
                    di = dst_v[ci, pl.ds(16 * g, 16)]
                    av = (plsc.load_gather(as_v, [si])
                          + plsc.load_gather(ad_v, [di]))
                    av = jnp.where(av >= 0.0, av, 0.2 * av)
                    av = jnp.minimum(av, 80.0)
                    ex_v[pl.ds(16 * g, 16)] = jnp.exp(av)

                pltpu.make_async_copy(
                    hs_hbm.at[pl.ds(0, CK)], rows_v.at[rb], gsem).wait()

                @plsc.parallel_loop(0, CK, 1, unroll=8)
                def _(e):
                    ei = jnp.full((16,), e, jnp.int32)
                    exv = plsc.load_gather(ex_v, [ei])
                    for j in range(nj):
                        rows_v[rb, e, pl.ds(16 * j, 16)] = (
                            rows_v[rb, e, pl.ds(16 * j, 16)] * exv)
                    sden_v[k2, e, pl.ds(0, 16)] = exv
                pltpu.async_copy(rows_v.at[rb], acc_sh.at[dst_v.at[ci]],
                                 ssem, add=True)
                pltpu.async_copy(sden_v.at[k2], den_sh.at[dst_v.at[ci]],
                                 ssem, add=True)
            return carry

        lax.fori_loop(0, NCH // 6, six_body, 0)
        for ci in (NCH - 2, NCH - 1):
            pltpu.make_async_copy(
                rows_v.at[ci % 3], acc_sh.at[pl.ds(0, CK)], ssem).wait()
            pltpu.make_async_copy(
                sden_v.at[ci % 2], den_sh.at[pl.ds(0, CK)], ssem).wait()
        plsc.subcore_barrier()

        nblk = ROWS_PT // 64
        pltpu.async_copy(acc_sh.at[pl.ds(row0, 64)], nbuf_v.at[0], gsem)
        pltpu.async_copy(den_sh.at[pl.ds(row0, 64)], dbuf_v.at[0], gsem)
        for b in range(nblk):
            k = b % 2
            if b + 1 < nblk:
                if b >= 1:
                    pltpu.make_async_copy(
                        nbuf_v.at[1 - k], out_hbm.at[c, pl.ds(0, 64)],
                        ssem).wait()
                pltpu.async_copy(acc_sh.at[pl.ds(row0 + 64 * (b + 1), 64)],
                                 nbuf_v.at[1 - k], gsem)
                pltpu.async_copy(den_sh.at[pl.ds(row0 + 64 * (b + 1), 64)],
                                 dbuf_v.at[1 - k], gsem)
            pltpu.make_async_copy(
                acc_sh.at[pl.ds(0, 64)], nbuf_v.at[k], gsem).wait()
            pltpu.make_async_copy(
                den_sh.at[pl.ds(0, 64)], dbuf_v.at[k], gsem).wait()

            @plsc.parallel_loop(0, 64, 1, unroll=8)
            def _(r):
                ri = jnp.full((16,), r, jnp.int32)
                dv = plsc.load_gather(
                    dbuf_v, [jnp.full((16,), k, jnp.int32), ri,
                             jnp.zeros((16,), jnp.int32)]) + EPS
                for j in range(nj):
                    nbuf_v[k, r, pl.ds(16 * j, 16)] = (
                        nbuf_v[k, r, pl.ds(16 * j, 16)] / dv)

            pltpu.async_copy(nbuf_v.at[k],
                             out_hbm.at[c, pl.ds(row0 + 64 * b, 64)], ssem)
        for b in (nblk - 2, nblk - 1):
            pltpu.make_async_copy(
                nbuf_v.at[b % 2], out_hbm.at[c, pl.ds(0, 64)], ssem).wait()

    return kern


_make_sc_edge_pass = functools.lru_cache(maxsize=None)(_make_sc_edge_pass)


def _prep_body(dout, fused, refs):
    if fused:
        (o0, o1, o2, bs3, wcat, wd, atts, attd, hs_ref, avec_ref) = refs
        i = pl.program_id(0)
        h = (o0[...] + o1[...] + o2[...]) * jnp.float32(1.0 / 3.0) + bs3[...]
        rows = lax.broadcasted_iota(jnp.int32, (RB, 1), 0) + i * RB
        h = jnp.where(rows < N, h, 0.0)
    else:
        (x_ref, wcat, wd, atts, attd, hs_ref, avec_ref) = refs
        h = x_ref[...]
    hs = jnp.dot(h, wcat[...], preferred_element_type=jnp.float32)
    hs_ref[...] = hs
    avs = []
    for t in range(3):
        hst = hs[:, t * dout:(t + 1) * dout]
        avs.append(jnp.sum(hst * atts[t][None, :], axis=-1))
    for t in range(3):
        wdv = jnp.dot(attd[t][None, :], wd[t],
                      preferred_element_type=jnp.float32)
        avs.append(jnp.sum(h * wdv, axis=-1))
    z = jnp.zeros_like(avs[0])
    avec_ref[...] = jnp.stack(avs + [z, z], axis=0)


def _make_tc_prep(dout, fused, win=128):
    grid = (NP // RB,)
    if fused:
        in_specs = [
            pl.BlockSpec((RB, win), lambda i: (i, 0)),
            pl.BlockSpec((RB, win), lambda i: (i, 0)),
            pl.BlockSpec((RB, win), lambda i: (i, 0)),
            pl.BlockSpec((1, win), lambda i: (0, 0)),
            pl.BlockSpec((128, 3 * dout), lambda i: (0, 0)),
            pl.BlockSpec((3, dout, 128), lambda i: (0, 0, 0)),
            pl.BlockSpec((3, dout), lambda i: (0, 0)),
            pl.BlockSpec((3, dout), lambda i: (0, 0)),
        ]
    else:
        in_specs = [
            pl.BlockSpec((RB, 128), lambda i: (i, 0)),
            pl.BlockSpec((128, 3 * dout), lambda i: (0, 0)),
            pl.BlockSpec((3, dout, 128), lambda i: (0, 0, 0)),
            pl.BlockSpec((3, dout), lambda i: (0, 0)),
            pl.BlockSpec((3, dout), lambda i: (0, 0)),
        ]
    out_specs = [
        pl.BlockSpec((RB, 3 * dout), lambda i: (i, 0)),
        pl.BlockSpec((8, RB), lambda i: (0, i)),
    ]
    out_shape = [
        jax.ShapeDtypeStruct((NP, 3 * dout), jnp.float32),
        jax.ShapeDtypeStruct((8, NP), jnp.float32),
    ]
    return pl.pallas_call(
        lambda *refs: _prep_body(dout, fused, refs),
        grid=grid, in_specs=in_specs, out_specs=out_specs,
        out_shape=out_shape)


def _final_body(o0, o1, o2, bs3, out_ref):
    out_ref[...] = (o0[...] + o1[...] + o2[...]) * jnp.float32(1.0 / 3.0) \
        + bs3[...]


_tc_final = pl.pallas_call(
    _final_body,
    grid=(NP // RB,),
    in_specs=[
        pl.BlockSpec((RB, 256), lambda i: (i, 0)),
        pl.BlockSpec((RB, 256), lambda i: (i, 0)),
        pl.BlockSpec((RB, 256), lambda i: (i, 0)),
        pl.BlockSpec((1, 256), lambda i: (0, 0)),
    ],
    out_specs=pl.BlockSpec((RB, 256), lambda i: (i, 0)),
    out_shape=jax.ShapeDtypeStruct((NP, 256), jnp.float32),
)


def _prep_edges(ei):
    s0, d0 = ei[0], ei[1]
    d0 = jnp.where(s0 != d0, d0, N)
    loop = jnp.arange(N, dtype=jnp.int32)
    src = jnp.concatenate([s0, loop])
    dst = jnp.concatenate([d0, loop])
    src = jnp.pad(src, (0, EPAD - E))
    dst = jnp.pad(dst, (0, EPAD - E), constant_values=N)
    src2 = jnp.stack([src, src + NP]).reshape(2, NSUB, NCH, CK)
    dst2 = dst.reshape(NSUB, NCH, CK)
    return src2, dst2


def _halves(hs_t, hw):
    return jnp.stack([hs_t[:, :hw], hs_t[:, hw:]]).reshape(2 * NP, hw)


def kernel(x, edge_index_0, edge_index_1, edge_index_2,
           Ws_0_0, Wd_0_0, atts_0_0, attd_0_0, b_0_0,
           Ws_0_1, Wd_0_1, atts_0_1, attd_0_1, b_0_1,
           Ws_0_2, Wd_0_2, atts_0_2, attd_0_2, b_0_2,
           Ws_1_0, Wd_1_0, atts_1_0, attd_1_0, b_1_0,
           Ws_1_1, Wd_1_1, atts_1_1, attd_1_1, b_1_1,
           Ws_1_2, Wd_1_2, atts_1_2, attd_1_2, b_1_2,
           Ws_2_0, Wd_2_0, atts_2_0, attd_2_0, b_2_0,
           Ws_2_1, Wd_2_1, atts_2_1, attd_2_1, b_2_1,
           Ws_2_2, Wd_2_2, atts_2_2, attd_2_2, b_2_2):
    p = dict(locals())
    edges = [_prep_edges(e) for e in (edge_index_0, edge_index_1, edge_index_2)]

    xp = jnp.pad(x, ((0, NP - N), (0, 0)))

    def layer_weights(l):
        dout = 256 if l == 2 else 128
        wcat = jnp.concatenate(
            [p[f"Ws_{l}_{t}"].T for t in range(3)], axis=1)
        wd = jnp.stack([p[f"Wd_{l}_{t}"] for t in range(3)])
        atts = jnp.stack([p[f"atts_{l}_{t}"] for t in range(3)])
        attd = jnp.stack([p[f"attd_{l}_{t}"] for t in range(3)])
        bs3 = ((p[f"b_{l}_0"] + p[f"b_{l}_1"] + p[f"b_{l}_2"])
               * jnp.float32(1.0 / 3.0))[None, :]
        return dout, wcat, wd, atts, attd, bs3

    outs = None
    prev_bs3 = None
    for l in range(3):
        dout, wcat, wd, atts, attd, bs3 = layer_weights(l)
        sc_pass = _make_sc_edge_pass(64)
        if l == 0:
            hs_cat, avec = _make_tc_prep(dout, False)(xp, wcat, wd, atts, attd)
        else:
            hs_cat, avec = _make_tc_prep(dout, True)(
                outs[0], outs[1], outs[2], prev_bs3, wcat, wd, atts, attd)
        outs = []
        for t in range(3):
            pieces = []
            for h in range(dout // 128):
                hs_t = hs_cat[:, t * dout + 128 * h:t * dout + 128 * (h + 1)]
                hsf = _halves(hs_t, 64)
                norm = sc_pass(edges[t][0], edges[t][1],
                               avec[t], avec[3 + t], hsf)
                pieces += [norm[0], norm[1]]
            outs.append(jnp.concatenate(pieces, axis=1))
        prev_bs3 = bs3
    out = _tc_final(outs[0], outs[1], outs[2], prev_bs3)
    return out[:N]
